# baseline (speedup 1.0000x reference)
import functools

import jax
import jax.numpy as jnp
from jax import lax
from jax.experimental import pallas as pl
from jax.experimental.pallas import tpu as pltpu
from jax.experimental.pallas import tpu_sc as plsc

NUM_CODES = 1024
EMBED_DIM = 256
BETA = 0.25
BLK = 1024


def _vq_body(z_ref, c_ref, idx_ref, d2_ref, csq_ref, iota_ref):
    cb = c_ref[...]

    @pl.when(pl.program_id(0) == 0)
    def _():
        csq_col = jnp.sum(cb * cb, axis=1, keepdims=True)
        csq_ref[...] = lax.transpose(csq_col, (1, 0))
        iota_ref[...] = lax.broadcasted_iota(
            jnp.int32, (BLK, NUM_CODES), 1).astype(jnp.float32)

    zb = z_ref[...]
    s2 = lax.dot_general(zb + zb, cb, (((1,), (1,)), ((), ())))
    zsq = jnp.sum(zb * zb, axis=1, keepdims=True)
    d2 = (zsq + csq_ref[...]) - s2
    d2 = jnp.maximum(d2, 0.0)
    dist = jnp.sqrt(d2)
    minval = jnp.min(dist, axis=1, keepdims=True)
    idxf = jnp.min(
        jnp.where(dist == minval, iota_ref[...], jnp.float32(NUM_CODES)),
        axis=1)
    idx_ref[0, 0, :] = idxf.astype(jnp.int32)
    d2_ref[0, 0, :] = (minval * minval)[:, 0]


def _st_body(q_ref, z_ref, d2m_ref, qst_ref, loss_ref):
    qb = q_ref[...]
    zb = z_ref[...]
    qst_ref[...] = zb + (qb - zb)

    @pl.when(pl.program_id(0) == 0)
    def _():
        n_elem = d2m_ref.shape[0] * d2m_ref.shape[2] * EMBED_DIM
        loss = (1.0 + BETA) * (jnp.sum(d2m_ref[...]) / n_elem)
        loss_ref[...] = loss.reshape(1, 1)


def _sc_gather(codebook, indices):
    info = plsc.get_sparse_core_info()
    nc, ns = info.num_cores, info.num_subcores
    nw = nc * ns
    n, d = indices.shape[0], codebook.shape[1]
    bpw = n // nw
    ch = 128
    nch = bpw // ch
    idx2 = indices.reshape(n // ch, ch)
    mesh = plsc.VectorSubcoreMesh(core_axis_name="c", subcore_axis_name="s")

    @functools.partial(
        pl.kernel, mesh=mesh,
        out_type=jax.ShapeDtypeStruct((n, d), jnp.float32),
        scratch_types=[
            pltpu.VMEM((ch,), jnp.int32),
            pltpu.VMEM((ch,), jnp.int32),
            pltpu.VMEM((bpw, d), jnp.float32),
            pltpu.SemaphoreType.DMA,
        ],
    )
    def gk(table_hbm, idx_hbm, out_hbm, ia, ib, rows, sem):
        w = lax.axis_index("s") * nc + lax.axis_index("c")
        r0 = w * nch
        pltpu.sync_copy(idx_hbm.at[r0], ia)
        pltpu.sync_copy(idx_hbm.at[r0 + 1], ib)
        c1 = pltpu.async_copy(table_hbm.at[ia], rows.at[pl.ds(0, ch)], sem)
        c2 = pltpu.async_copy(table_hbm.at[ib], rows.at[pl.ds(ch, ch)], sem)
        c1.wait()
        c2.wait()
        pltpu.sync_copy(rows, out_hbm.at[pl.ds(w * bpw, bpw)])

    return gk(codebook, idx2)


def kernel(z, codebook):
    B, D, H, W = z.shape
    N = B * H * W
    M = codebook.shape[0]
    G = N // BLK

    z_flat = jnp.transpose(z, (0, 2, 3, 1)).reshape(N, D)

    idx3, d2m = pl.pallas_call(
        _vq_body,
        grid=(G,),
        in_specs=[
            pl.BlockSpec((BLK, D), lambda i: (i, 0)),
            pl.BlockSpec((M, D), lambda i: (0, 0)),
        ],
        out_specs=[
            pl.BlockSpec((1, 1, BLK), lambda i: (i, 0, 0)),
            pl.BlockSpec((1, 1, BLK), lambda i: (i, 0, 0)),
        ],
        out_shape=[
            jax.ShapeDtypeStruct((G, 1, BLK), jnp.int32),
            jax.ShapeDtypeStruct((G, 1, BLK), jnp.float32),
        ],
        scratch_shapes=[pltpu.VMEM((1, M), jnp.float32),
                        pltpu.VMEM((BLK, M), jnp.float32)],
    )(z_flat, codebook)

    indices = idx3.reshape(N)
    q = _sc_gather(codebook, indices)

    qst, loss11 = pl.pallas_call(
        _st_body,
        grid=(G,),
        in_specs=[
            pl.BlockSpec((BLK, D), lambda i: (i, 0)),
            pl.BlockSpec((BLK, D), lambda i: (i, 0)),
            pl.BlockSpec((G, 1, BLK), lambda i: (0, 0, 0)),
        ],
        out_specs=[
            pl.BlockSpec((BLK, D), lambda i: (i, 0)),
            pl.BlockSpec((1, 1), lambda i: (0, 0)),
        ],
        out_shape=[
            jax.ShapeDtypeStruct((N, D), jnp.float32),
            jax.ShapeDtypeStruct((1, 1), jnp.float32),
        ],
    )(q, z_flat, d2m)

    z_q = jnp.transpose(qst.reshape(B, H, W, D), (0, 3, 1, 2))
    vq_loss = loss11.reshape(())
    return z_q, vq_loss, indices.reshape(B, H, W)


# speedup vs baseline: 1.2401x; 1.2401x over previous
import functools

import jax
import jax.numpy as jnp
from jax import lax
from jax.experimental import pallas as pl
from jax.experimental.pallas import tpu as pltpu
from jax.experimental.pallas import tpu_sc as plsc

NUM_CODES = 1024
EMBED_DIM = 256
BETA = 0.25
BLK = 1024


def _vq_body(z_ref, c_ref, idx_ref, loss_ref, csq_ref, iota_ref):
    cb = c_ref[...]
    i = pl.program_id(0)

    @pl.when(i == 0)
    def _():
        csq_col = jnp.sum(cb * cb, axis=1, keepdims=True)
        csq_ref[...] = lax.transpose(csq_col, (1, 0))
        iota_ref[...] = lax.broadcasted_iota(
            jnp.int32, (BLK, NUM_CODES), 1).astype(jnp.float32)
        loss_ref[...] = jnp.zeros((1, 1), jnp.float32)

    zb = z_ref[...]
    s2 = lax.dot_general(zb + zb, cb, (((1,), (1,)), ((), ())))
    zsq = jnp.sum(zb * zb, axis=1, keepdims=True)
    d2 = (zsq + csq_ref[...]) - s2
    d2 = jnp.maximum(d2, 0.0)
    dist = jnp.sqrt(d2)
    minval = jnp.min(dist, axis=1, keepdims=True)
    idxf = jnp.min(
        jnp.where(dist == minval, iota_ref[...], jnp.float32(NUM_CODES)),
        axis=1)
    idx_ref[0, 0, :] = idxf.astype(jnp.int32)
    loss_ref[...] += jnp.sum(minval * minval).reshape(1, 1)

    @pl.when(i == pl.num_programs(0) - 1)
    def _():
        n_elem = pl.num_programs(0) * BLK * EMBED_DIM
        loss_ref[...] = (1.0 + BETA) * (loss_ref[...] / n_elem)


def _sc_gather(codebook, indices):
    info = plsc.get_sparse_core_info()
    nc, ns = info.num_cores, info.num_subcores
    nw = nc * ns
    n, d = indices.shape[0], codebook.shape[1]
    bpw = n // nw
    ch = 128
    nch = bpw // ch
    idx2 = indices.reshape(n // ch, ch)
    mesh = plsc.VectorSubcoreMesh(core_axis_name="c", subcore_axis_name="s")

    @functools.partial(
        pl.kernel, mesh=mesh,
        out_type=jax.ShapeDtypeStruct((n, d), jnp.float32),
        scratch_types=[
            pltpu.VMEM((ch,), jnp.int32),
            pltpu.VMEM((ch,), jnp.int32),
            pltpu.VMEM((bpw, d), jnp.float32),
            pltpu.SemaphoreType.DMA,
        ],
    )
    def gk(table_hbm, idx_hbm, out_hbm, ia, ib, rows, sem):
        w = lax.axis_index("s") * nc + lax.axis_index("c")
        r0 = w * nch
        pltpu.sync_copy(idx_hbm.at[r0], ia)
        pltpu.sync_copy(idx_hbm.at[r0 + 1], ib)
        c1 = pltpu.async_copy(table_hbm.at[ia], rows.at[pl.ds(0, ch)], sem)
        c2 = pltpu.async_copy(table_hbm.at[ib], rows.at[pl.ds(ch, ch)], sem)
        c1.wait()
        c2.wait()
        pltpu.sync_copy(rows, out_hbm.at[pl.ds(w * bpw, bpw)])

    return gk(codebook, idx2)


def kernel(z, codebook):
    B, D, H, W = z.shape
    N = B * H * W
    M = codebook.shape[0]
    G = N // BLK

    z_flat = jnp.transpose(z, (0, 2, 3, 1)).reshape(N, D)

    idx3, loss11 = pl.pallas_call(
        _vq_body,
        grid=(G,),
        in_specs=[
            pl.BlockSpec((BLK, D), lambda i: (i, 0)),
            pl.BlockSpec((M, D), lambda i: (0, 0)),
        ],
        out_specs=[
            pl.BlockSpec((1, 1, BLK), lambda i: (i, 0, 0)),
            pl.BlockSpec((1, 1), lambda i: (0, 0)),
        ],
        out_shape=[
            jax.ShapeDtypeStruct((G, 1, BLK), jnp.int32),
            jax.ShapeDtypeStruct((1, 1), jnp.float32),
        ],
        scratch_shapes=[pltpu.VMEM((1, M), jnp.float32),
                        pltpu.VMEM((BLK, M), jnp.float32)],
    )(z_flat, codebook)

    indices = idx3.reshape(N)
    q = _sc_gather(codebook, indices)

    z_q = jnp.transpose(q.reshape(B, H, W, D), (0, 3, 1, 2))
    vq_loss = loss11.reshape(())
    return z_q, vq_loss, indices.reshape(B, H, W)


# speedup vs baseline: 1.3844x; 1.1164x over previous
import functools

import jax
import jax.numpy as jnp
from jax import lax
from jax.experimental import pallas as pl
from jax.experimental.pallas import tpu as pltpu
from jax.experimental.pallas import tpu_sc as plsc

NUM_CODES = 1024
EMBED_DIM = 256
BETA = 0.25
BLK = 1024


def _vq_body(z_ref, c_ref, idx_ref, loss_ref, csq_ref, iota_ref):
    cb = c_ref[...]
    i = pl.program_id(0)

    @pl.when(i == 0)
    def _():
        csq_col = jnp.sum(cb * cb, axis=1, keepdims=True)
        csq_ref[...] = lax.transpose(csq_col, (1, 0))
        iota_ref[...] = lax.broadcasted_iota(
            jnp.int32, (BLK, NUM_CODES), 1).astype(jnp.float32)
        loss_ref[...] = jnp.zeros((1, 1), jnp.float32)

    zb = z_ref[...]
    s2 = lax.dot_general(zb + zb, cb, (((1,), (1,)), ((), ())))
    zsq = jnp.sum(zb * zb, axis=1, keepdims=True)
    d2 = (zsq + csq_ref[...]) - s2
    d2 = jnp.maximum(d2, 0.0)
    dist = jnp.sqrt(d2)
    minval = jnp.min(dist, axis=1, keepdims=True)
    idxf = jnp.min(
        jnp.where(dist == minval, iota_ref[...], jnp.float32(NUM_CODES)),
        axis=1)
    idx_ref[...] = idxf.astype(jnp.int32).reshape(idx_ref.shape)
    loss_ref[...] += jnp.sum(minval * minval).reshape(1, 1)

    @pl.when(i == pl.num_programs(0) - 1)
    def _():
        n_elem = pl.num_programs(0) * BLK * EMBED_DIM
        loss_ref[...] = (1.0 + BETA) * (loss_ref[...] / n_elem)


def _sc_gather(codebook, indices):
    info = plsc.get_sparse_core_info()
    nc, ns = info.num_cores, info.num_subcores
    nw = nc * ns
    n, d = indices.shape[0], codebook.shape[1]
    bpw = n // nw
    ch = 128
    nch = bpw // ch
    idx2 = indices.reshape(n // ch, ch)
    mesh = plsc.VectorSubcoreMesh(core_axis_name="c", subcore_axis_name="s")

    @functools.partial(
        pl.kernel, mesh=mesh,
        out_type=jax.ShapeDtypeStruct((n, d), jnp.float32),
        scratch_types=[
            pltpu.VMEM((ch,), jnp.int32),
            pltpu.VMEM((ch,), jnp.int32),
            pltpu.VMEM((bpw, d), jnp.float32),
            pltpu.SemaphoreType.DMA,
        ],
    )
    def gk(table_hbm, idx_hbm, out_hbm, ia, ib, rows, sem):
        w = lax.axis_index("s") * nc + lax.axis_index("c")
        r0 = w * nch
        pltpu.sync_copy(idx_hbm.at[r0], ia)
        c1 = pltpu.async_copy(table_hbm.at[ia], rows.at[pl.ds(0, ch)], sem)
        pltpu.sync_copy(idx_hbm.at[r0 + 1], ib)
        c2 = pltpu.async_copy(table_hbm.at[ib], rows.at[pl.ds(ch, ch)], sem)
        c1.wait()
        pltpu.sync_copy(rows.at[pl.ds(0, ch)],
                        out_hbm.at[pl.ds(w * bpw, ch)])
        c2.wait()
        pltpu.sync_copy(rows.at[pl.ds(ch, ch)],
                        out_hbm.at[pl.ds(w * bpw + ch, ch)])

    return gk(codebook, idx2)


def kernel(z, codebook):
    B, D, H, W = z.shape
    N = B * H * W
    M = codebook.shape[0]
    G = N // BLK

    z_flat = jnp.transpose(z, (0, 2, 3, 1)).reshape(N, D)

    rows_per_blk = BLK // 128
    idx2, loss11 = pl.pallas_call(
        _vq_body,
        grid=(G,),
        in_specs=[
            pl.BlockSpec((BLK, D), lambda i: (i, 0)),
            pl.BlockSpec((M, D), lambda i: (0, 0)),
        ],
        out_specs=[
            pl.BlockSpec((rows_per_blk, 128), lambda i: (i, 0)),
            pl.BlockSpec((1, 1), lambda i: (0, 0)),
        ],
        out_shape=[
            jax.ShapeDtypeStruct((N // 128, 128), jnp.int32),
            jax.ShapeDtypeStruct((1, 1), jnp.float32),
        ],
        scratch_shapes=[pltpu.VMEM((1, M), jnp.float32),
                        pltpu.VMEM((BLK, M), jnp.float32)],
    )(z_flat, codebook)

    q = _sc_gather(codebook, idx2.reshape(N))

    z_q = jnp.transpose(q.reshape(B, H, W, D), (0, 3, 1, 2))
    vq_loss = loss11.reshape(())
    return z_q, vq_loss, idx2.reshape(B, H, W)


# speedup vs baseline: 1.3957x; 1.0081x over previous
import functools

import jax
import jax.numpy as jnp
from jax import lax
from jax.experimental import pallas as pl
from jax.experimental.pallas import tpu as pltpu
from jax.experimental.pallas import tpu_sc as plsc

NUM_CODES = 1024
EMBED_DIM = 256
BETA = 0.25
BLK = 2048


def _vq_body(z_ref, c_ref, idx_ref, loss_ref, csq_ref, iota_ref):
    cb = c_ref[...]
    i = pl.program_id(0)

    @pl.when(i == 0)
    def _():
        csq_col = jnp.sum(cb * cb, axis=1, keepdims=True)
        csq_ref[...] = lax.transpose(csq_col, (1, 0))
        iota_ref[...] = lax.broadcasted_iota(
            jnp.int32, (BLK, NUM_CODES), 1).astype(jnp.float32)
        loss_ref[...] = jnp.zeros((1, 1), jnp.float32)

    zb = z_ref[...]
    s2 = lax.dot_general(zb + zb, cb, (((1,), (1,)), ((), ())))
    zsq = jnp.sum(zb * zb, axis=1, keepdims=True)
    d2 = (zsq + csq_ref[...]) - s2
    d2 = jnp.maximum(d2, 0.0)
    dist = jnp.sqrt(d2)
    minval = jnp.min(dist, axis=1, keepdims=True)
    idxf = jnp.min(
        jnp.where(dist == minval, iota_ref[...], jnp.float32(NUM_CODES)),
        axis=1)
    idx_ref[...] = idxf.astype(jnp.int32).reshape(idx_ref.shape)
    loss_ref[...] += jnp.sum(minval * minval).reshape(1, 1)

    @pl.when(i == pl.num_programs(0) - 1)
    def _():
        n_elem = pl.num_programs(0) * BLK * EMBED_DIM
        loss_ref[...] = (1.0 + BETA) * (loss_ref[...] / n_elem)


def _sc_gather(codebook, indices):
    info = plsc.get_sparse_core_info()
    nc, ns = info.num_cores, info.num_subcores
    nw = nc * ns
    n, d = indices.shape[0], codebook.shape[1]
    bpw = n // nw
    ch = 128
    nch = bpw // ch
    idx2 = indices.reshape(n // ch, ch)
    mesh = plsc.VectorSubcoreMesh(core_axis_name="c", subcore_axis_name="s")

    @functools.partial(
        pl.kernel, mesh=mesh,
        out_type=jax.ShapeDtypeStruct((n, d), jnp.float32),
        scratch_types=[
            pltpu.VMEM((ch,), jnp.int32),
            pltpu.VMEM((ch,), jnp.int32),
            pltpu.VMEM((bpw, d), jnp.float32),
            pltpu.SemaphoreType.DMA,
        ],
    )
    def gk(table_hbm, idx_hbm, out_hbm, ia, ib, rows, sem):
        w = lax.axis_index("s") * nc + lax.axis_index("c")
        r0 = w * nch
        pltpu.sync_copy(idx_hbm.at[r0], ia)
        c1 = pltpu.async_copy(table_hbm.at[ia], rows.at[pl.ds(0, ch)], sem)
        pltpu.sync_copy(idx_hbm.at[r0 + 1], ib)
        c2 = pltpu.async_copy(table_hbm.at[ib], rows.at[pl.ds(ch, ch)], sem)
        c1.wait()
        pltpu.sync_copy(rows.at[pl.ds(0, ch)],
                        out_hbm.at[pl.ds(w * bpw, ch)])
        c2.wait()
        pltpu.sync_copy(rows.at[pl.ds(ch, ch)],
                        out_hbm.at[pl.ds(w * bpw + ch, ch)])

    return gk(codebook, idx2)


def kernel(z, codebook):
    B, D, H, W = z.shape
    N = B * H * W
    M = codebook.shape[0]
    G = N // BLK

    z_flat = jnp.transpose(z, (0, 2, 3, 1)).reshape(N, D)

    rows_per_blk = BLK // 128
    idx2, loss11 = pl.pallas_call(
        _vq_body,
        grid=(G,),
        in_specs=[
            pl.BlockSpec((BLK, D), lambda i: (i, 0)),
            pl.BlockSpec((M, D), lambda i: (0, 0)),
        ],
        out_specs=[
            pl.BlockSpec((rows_per_blk, 128), lambda i: (i, 0)),
            pl.BlockSpec((1, 1), lambda i: (0, 0)),
        ],
        out_shape=[
            jax.ShapeDtypeStruct((N // 128, 128), jnp.int32),
            jax.ShapeDtypeStruct((1, 1), jnp.float32),
        ],
        scratch_shapes=[pltpu.VMEM((1, M), jnp.float32),
                        pltpu.VMEM((BLK, M), jnp.float32)],
    )(z_flat, codebook)

    q = _sc_gather(codebook, idx2.reshape(N))

    z_q = jnp.transpose(q.reshape(B, H, W, D), (0, 3, 1, 2))
    vq_loss = loss11.reshape(())
    return z_q, vq_loss, idx2.reshape(B, H, W)
